# baseline (speedup 1.0000x reference)
import jax
import jax.numpy as jnp
from jax.experimental import pallas as pl
from jax.experimental.pallas import tpu as pltpu

_TILE = 1024


def _body(h_ref, W1_ref, b1_ref, Wa_ref, ba_ref, Wb_ref, bb_ref, Wc_ref,
          bc_ref, Wcls_ref, bcls_ref, fs_ref, yprob_ref, yhat_ref, det_ref):
    i = pl.program_id(0)
    nsteps = pl.num_programs(0)
    f32 = jnp.float32
    dn = (((0,), (0,)), ((), ()))
    tile = h_ref.shape[0]
    ones_row = jnp.ones((1, tile), f32)

    x = jnp.dot(h_ref[...], W1_ref[...], preferred_element_type=f32)
    x = x + b1_ref[...][None, :]
    xT = jnp.maximum(x.T, 0.0)
    xTa = jnp.concatenate([xT, ones_row], axis=0)
    Waa = jnp.concatenate([Wa_ref[...], ba_ref[...][None, :]], axis=0)
    Wba = jnp.concatenate([Wb_ref[...], bb_ref[...][None, :]], axis=0)
    aT = jnp.tanh(
        jax.lax.dot_general(Waa, xTa, dn, preferred_element_type=f32))
    bT = jax.nn.sigmoid(
        jax.lax.dot_general(Wba, xTa, dn, preferred_element_type=f32))
    abTa = jnp.concatenate([aT * bT, ones_row], axis=0)
    Wca = jnp.concatenate([Wc_ref[...], bc_ref[...][None, :]], axis=0)
    Wclsa = jnp.concatenate([Wcls_ref[...], bcls_ref[...][None, :]], axis=0)
    det = jax.lax.dot_general(Wca, abTa, dn, preferred_element_type=f32)
    cls = jax.lax.dot_general(Wclsa, xTa, dn, preferred_element_type=f32)
    cm = jnp.max(cls, axis=0, keepdims=True)
    ce = jnp.exp(cls - cm)
    cls_score = ce / jnp.sum(ce, axis=0, keepdims=True)

    det_ref[:, pl.ds(i * tile, tile)] = det
    fs_ref[:, pl.ds(i * tile, tile)] = cls_score

    @pl.when(i == nsteps - 1)
    def _finalize():
        d = det_ref[...]
        m = jnp.max(d, axis=1, keepdims=True)
        e = jnp.exp(d - m)
        s = jnp.sum(e, axis=1, keepdims=True)
        fs = fs_ref[...] * (e / s)
        fs_ref[...] = fs
        lo, hi = 1e-10, 1.0 - 1e-10
        yp0 = jnp.clip(jnp.sum(fs[0:1, :]), lo, hi)
        yp1 = jnp.clip(jnp.sum(fs[1:2, :]), lo, hi)
        yprob_ref[0, 0] = yp0
        yprob_ref[0, 1] = yp1
        yhat_ref[0, 0] = jnp.where(yp1 > yp0, 1, 0).astype(jnp.int32)


def kernel(h, W1, b1, Wa, ba, Wb, bb, Wc, bc, Wcls, bcls):
    N, FEA = h.shape
    H = W1.shape[1]
    D = Wa.shape[1]
    C = Wc.shape[1]
    tile = _TILE
    grid = (N // tile,)

    full = lambda shape: pl.BlockSpec(shape, lambda i: tuple(0 for _ in shape))
    fsT, ypr, yh = pl.pallas_call(
        _body,
        grid=grid,
        in_specs=[
            pl.BlockSpec((tile, FEA), lambda i: (i, 0)),
            full((FEA, H)),
            full((H,)),
            full((H, D)),
            full((D,)),
            full((H, D)),
            full((D,)),
            full((D, C)),
            full((C,)),
            full((H, C)),
            full((C,)),
        ],
        out_specs=[
            pl.BlockSpec((C, N), lambda i: (0, 0)),
            pl.BlockSpec(memory_space=pltpu.SMEM),
            pl.BlockSpec(memory_space=pltpu.SMEM),
        ],
        out_shape=[
            jax.ShapeDtypeStruct((C, N), jnp.float32),
            jax.ShapeDtypeStruct((1, C), jnp.float32),
            jax.ShapeDtypeStruct((1, 1), jnp.int32),
        ],
        scratch_shapes=[pltpu.VMEM((C, N), jnp.float32)],
        compiler_params=pltpu.CompilerParams(
            dimension_semantics=("arbitrary",)),
    )(h, W1, b1, Wa, ba, Wb, bb, Wc, bc, Wcls, bcls)

    return (fsT.T, ypr.reshape(C), yh.reshape(1))


# speedup vs baseline: 1.1919x; 1.1919x over previous
import jax
import jax.numpy as jnp
from jax.experimental import pallas as pl
from jax.experimental.pallas import tpu as pltpu

_TILE = 2048


def _body(h0_ref, h1_ref, W1_ref, b1_ref, Wa_ref, ba_ref, Wb_ref, bb_ref,
          Wc_ref, bc_ref, Wcls_ref, bcls_ref, fs_ref, yprob_ref, yhat_ref,
          det_ref):
    i = pl.program_id(0)
    nsteps = pl.num_programs(0)
    f32 = jnp.float32
    dn = (((0,), (0,)), ((), ()))
    tile = h0_ref.shape[0]
    half = h0_ref.shape[1]
    ones_row = jnp.ones((1, tile), f32)

    x = (jnp.dot(h0_ref[...], W1_ref[0:half, :], preferred_element_type=f32)
         + jnp.dot(h1_ref[...], W1_ref[half:, :], preferred_element_type=f32))
    x = x + b1_ref[...][None, :]
    xT = jnp.maximum(x.T, 0.0)
    xTa = jnp.concatenate([xT, ones_row], axis=0)
    Wfuse = jnp.concatenate([
        jnp.concatenate([Wa_ref[...], ba_ref[...][None, :]], axis=0),
        jnp.concatenate([Wb_ref[...], bb_ref[...][None, :]], axis=0),
        jnp.concatenate([Wcls_ref[...], bcls_ref[...][None, :]], axis=0),
    ], axis=1)
    D = Wa_ref.shape[1]
    y = jax.lax.dot_general(Wfuse, xTa, dn, preferred_element_type=f32)
    aT = jnp.tanh(y[0:D, :])
    bT = jax.nn.sigmoid(y[D:2 * D, :])
    cls = y[2 * D:, :]
    abTa = jnp.concatenate([aT * bT, ones_row], axis=0)
    Wca = jnp.concatenate([Wc_ref[...], bc_ref[...][None, :]], axis=0)
    det = jax.lax.dot_general(Wca, abTa, dn, preferred_element_type=f32)
    cm = jnp.max(cls, axis=0, keepdims=True)
    ce = jnp.exp(cls - cm)
    cls_score = ce / jnp.sum(ce, axis=0, keepdims=True)

    det_ref[:, pl.ds(i * tile, tile)] = det
    fs_ref[:, pl.ds(i * tile, tile)] = cls_score

    @pl.when(i == nsteps - 1)
    def _finalize():
        d = det_ref[...]
        m = jnp.max(d, axis=1, keepdims=True)
        e = jnp.exp(d - m)
        s = jnp.sum(e, axis=1, keepdims=True)
        fs = fs_ref[...] * (e / s)
        fs_ref[...] = fs
        lo, hi = 1e-10, 1.0 - 1e-10
        yp0 = jnp.clip(jnp.sum(fs[0:1, :]), lo, hi)
        yp1 = jnp.clip(jnp.sum(fs[1:2, :]), lo, hi)
        yprob_ref[0, 0] = yp0
        yprob_ref[0, 1] = yp1
        yhat_ref[0, 0] = jnp.where(yp1 > yp0, 1, 0).astype(jnp.int32)


def kernel(h, W1, b1, Wa, ba, Wb, bb, Wc, bc, Wcls, bcls):
    N, FEA = h.shape
    H = W1.shape[1]
    D = Wa.shape[1]
    C = Wc.shape[1]
    tile = _TILE
    half = FEA // 2
    grid = (N // tile,)

    full = lambda shape: pl.BlockSpec(shape, lambda i: tuple(0 for _ in shape))
    fsT, ypr, yh = pl.pallas_call(
        _body,
        grid=grid,
        in_specs=[
            pl.BlockSpec((tile, half), lambda i: (i, 0)),
            pl.BlockSpec((tile, half), lambda i: (i, 1)),
            full((FEA, H)),
            full((H,)),
            full((H, D)),
            full((D,)),
            full((H, D)),
            full((D,)),
            full((D, C)),
            full((C,)),
            full((H, C)),
            full((C,)),
        ],
        out_specs=[
            pl.BlockSpec((C, N), lambda i: (0, 0)),
            pl.BlockSpec(memory_space=pltpu.SMEM),
            pl.BlockSpec(memory_space=pltpu.SMEM),
        ],
        out_shape=[
            jax.ShapeDtypeStruct((C, N), jnp.float32),
            jax.ShapeDtypeStruct((1, C), jnp.float32),
            jax.ShapeDtypeStruct((1, 1), jnp.int32),
        ],
        scratch_shapes=[pltpu.VMEM((C, N), jnp.float32)],
        compiler_params=pltpu.CompilerParams(
            dimension_semantics=("arbitrary",)),
    )(h, h, W1, b1, Wa, ba, Wb, bb, Wc, bc, Wcls, bcls)

    return (fsT.T, ypr.reshape(C), yh.reshape(1))
